# baseline (speedup 1.0000x reference)
import jax
import jax.numpy as jnp
from jax import lax
from jax.experimental import pallas as pl
from jax.experimental.pallas import tpu as pltpu
from jax.experimental.pallas import tpu_sc as plsc

NC = 2
NS = 16
NW = NC * NS
GE = 512
FEAT = 16
PACK = 8

STRIPE = 6256
ACC_ROWS = STRIPE * NS
ACC_PACK = ACC_ROWS // PACK


def _mesh():
    return plsc.VectorSubcoreMesh(core_axis_name="c", subcore_axis_name="s")


def _groups_per_tile(n_edges):
    g = -(-n_edges // (NW * GE))
    return -(-g // 6) * 6


def _acc_out_spec():
    return (jax.ShapeDtypeStruct((ACC_ROWS, FEAT), jnp.float32),
            jax.ShapeDtypeStruct((ACC_ROWS, FEAT), jnp.float32))


def _zero_and_barrier(z_hbm, acc_sh, s):
    pltpu.sync_copy(z_hbm.at[pl.ds(s * STRIPE, STRIPE)],
                    acc_sh.at[pl.ds(s * STRIPE, STRIPE)])


def _copy_out(acc_sh, out0, out1, c, s):
    @pl.when(c == 0)
    def _():
        pltpu.sync_copy(acc_sh.at[pl.ds(s * STRIPE, STRIPE)],
                        out0.at[pl.ds(s * STRIPE, STRIPE)])

    @pl.when(c == 1)
    def _():
        pltpu.sync_copy(acc_sh.at[pl.ds(s * STRIPE, STRIPE)],
                        out1.at[pl.ds(s * STRIPE, STRIPE)])


def _sc_degree(dst_flat, zeros_f, ones_blk, gpt):
    def body(dst_hbm, z_hbm, one_hbm, out0, out1, dstv0, dstv1, onev,
             acc_sh, ss0, ss1):
        c = lax.axis_index("c")
        s = lax.axis_index("s")
        _zero_and_barrier(z_hbm, acc_sh, s)
        pltpu.sync_copy(one_hbm, onev)
        plsc.subcore_barrier()
        base = (c * NS + s) * gpt
        dstv = (dstv0, dstv1)
        ss = (ss0, ss1)

        def steps(g, b):
            @pl.when(g >= 2)
            def _():
                pltpu.make_async_copy(onev, acc_sh.at[dstv[b]], ss[b]).wait()
            pltpu.sync_copy(dst_hbm.at[pl.ds((base + g) * GE, GE)], dstv[b])
            pltpu.async_copy(onev, acc_sh.at[dstv[b]], ss[b], add=True)

        def pair(i, carry):
            steps(2 * i, 0)
            steps(2 * i + 1, 1)
            return carry

        lax.fori_loop(0, gpt // 2, pair, 0)
        pltpu.make_async_copy(onev, acc_sh.at[dstv0], ss0).wait()
        pltpu.make_async_copy(onev, acc_sh.at[dstv1], ss1).wait()
        plsc.subcore_barrier()
        _copy_out(acc_sh, out0, out1, c, s)

    f = pl.kernel(
        body,
        out_type=_acc_out_spec(),
        mesh=_mesh(),
        scratch_types=[
            pltpu.VMEM((GE,), jnp.int32),
            pltpu.VMEM((GE,), jnp.int32),
            pltpu.VMEM((GE, FEAT), jnp.float32),
            pltpu.VMEM_SHARED((ACC_ROWS, FEAT), jnp.float32),
            pltpu.SemaphoreType.DMA,
            pltpu.SemaphoreType.DMA,
        ],
        compiler_params=pltpu.CompilerParams(use_tc_tiling_on_sc=False),
    )
    return f(dst_flat, zeros_f, ones_blk)


def _sc_aggregate(edges_grp, p, zeros_f, gpt):
    def body(e_hbm, p_hbm, z_hbm, out0, out1, sd0, sd1, sd2, rows0, rows1,
             rows2, acc_sh, gs0, gs1, gs2, ss0, ss1, ss2, is0, is1, is2):
        c = lax.axis_index("c")
        s = lax.axis_index("s")
        _zero_and_barrier(z_hbm, acc_sh, s)
        plsc.subcore_barrier()
        base = (c * NS + s) * gpt
        sd = (sd0, sd1, sd2)
        rows = (rows0, rows1, rows2)
        gs = (gs0, gs1, gs2)
        ss = (ss0, ss1, ss2)
        isem = (is0, is1, is2)

        def steps(g, b):
            pb = (b + 1) % 3
            @pl.when(g >= 2)
            def _():
                pltpu.make_async_copy(rows[pb], acc_sh.at[sd[pb].at[1]],
                                      ss[pb]).wait()

            @pl.when(g + 1 < gpt)
            def _():
                pltpu.async_copy(e_hbm.at[base + g + 1], sd[pb], isem[pb])
            pltpu.make_async_copy(p_hbm.at[sd[b].at[0]], rows[b],
                                  gs[b]).wait()
            pltpu.async_copy(rows[b], acc_sh.at[sd[b].at[1]], ss[b],
                             add=True)

            @pl.when(g + 1 < gpt)
            def _():
                pltpu.make_async_copy(e_hbm.at[base + g + 1], sd[pb],
                                      isem[pb]).wait()
                pltpu.async_copy(p_hbm.at[sd[pb].at[0]], rows[pb], gs[pb])

        def triple(i, carry):
            steps(3 * i, 0)
            steps(3 * i + 1, 1)
            steps(3 * i + 2, 2)
            return carry

        pltpu.sync_copy(e_hbm.at[base], sd0)
        pltpu.async_copy(p_hbm.at[sd0.at[0]], rows0, gs0)
        lax.fori_loop(0, gpt // 3, triple, 0)
        b_last = (gpt - 1) % 3
        b_prev = (gpt - 2) % 3
        pltpu.make_async_copy(rows[b_prev], acc_sh.at[sd[b_prev].at[1]],
                              ss[b_prev]).wait()
        pltpu.make_async_copy(rows[b_last], acc_sh.at[sd[b_last].at[1]],
                              ss[b_last]).wait()
        plsc.subcore_barrier()
        _copy_out(acc_sh, out0, out1, c, s)

    f = pl.kernel(
        body,
        out_type=_acc_out_spec(),
        mesh=_mesh(),
        scratch_types=[
            pltpu.VMEM((2, GE), jnp.int32),
            pltpu.VMEM((2, GE), jnp.int32),
            pltpu.VMEM((2, GE), jnp.int32),
            pltpu.VMEM((GE, FEAT), jnp.float32),
            pltpu.VMEM((GE, FEAT), jnp.float32),
            pltpu.VMEM((GE, FEAT), jnp.float32),
            pltpu.VMEM_SHARED((ACC_ROWS, FEAT), jnp.float32),
            pltpu.SemaphoreType.DMA,
            pltpu.SemaphoreType.DMA,
            pltpu.SemaphoreType.DMA,
            pltpu.SemaphoreType.DMA,
            pltpu.SemaphoreType.DMA,
            pltpu.SemaphoreType.DMA,
            pltpu.SemaphoreType.DMA,
            pltpu.SemaphoreType.DMA,
            pltpu.SemaphoreType.DMA,
        ],
        compiler_params=pltpu.CompilerParams(use_tc_tiling_on_sc=False),
    )
    return f(edges_grp, p, zeros_f)


_PBLK = 1024


def _pgrid():
    return (-(-ACC_PACK // _PBLK),)


def _pspec():
    return pl.BlockSpec((_PBLK, 128), lambda i: (i, 0))


def _fspec(r, c):
    return pl.BlockSpec((r, c), lambda i: (0, 0))


def _stage_a_body(a0, a1, xp, wb, bmat, d_out, p_out):
    deg = jnp.dot(a0[...] + a1[...], bmat[...],
                  preferred_element_type=jnp.float32) + 1.0
    dd = lax.rsqrt(deg)
    d_out[...] = dd
    xw = jnp.dot(xp[...], wb[...], preferred_element_type=jnp.float32)
    p_out[...] = dd * xw


def _stage_mid_body(a0, a1, p, d, bt, wb, out):
    dd = d[...]
    h = dd * (a0[...] + a1[...] + p[...]) + bt[...]
    out[...] = dd * jnp.dot(jnp.tanh(h), wb[...],
                            preferred_element_type=jnp.float32)


def _stage_last_body(a0, a1, p, d, bt, out):
    out[...] = d[...] * (a0[...] + a1[...] + p[...]) + bt[...]


def _tc_stage_a(a0p, a1p, xp, w1b, bmat):
    return pl.pallas_call(
        _stage_a_body,
        grid=_pgrid(),
        in_specs=[_pspec(), _pspec(), _pspec(),
                  _fspec(128, 128), _fspec(128, 128)],
        out_specs=[_pspec(), _pspec()],
        out_shape=[jax.ShapeDtypeStruct((ACC_PACK, 128), jnp.float32),
                   jax.ShapeDtypeStruct((ACC_PACK, 128), jnp.float32)],
    )(a0p, a1p, xp, w1b, bmat)


def _tc_stage_mid(a0p, a1p, pp, dp, bt, wbig):
    return pl.pallas_call(
        _stage_mid_body,
        grid=_pgrid(),
        in_specs=[_pspec(), _pspec(), _pspec(), _pspec(),
                  _fspec(1, 128), _fspec(128, 128)],
        out_specs=_pspec(),
        out_shape=jax.ShapeDtypeStruct((ACC_PACK, 128), jnp.float32),
    )(a0p, a1p, pp, dp, bt, wbig)


def _tc_stage_last(a0p, a1p, pp, dp, bt):
    return pl.pallas_call(
        _stage_last_body,
        grid=_pgrid(),
        in_specs=[_pspec(), _pspec(), _pspec(), _pspec(), _fspec(1, 128)],
        out_specs=_pspec(),
        out_shape=jax.ShapeDtypeStruct((ACC_PACK, 128), jnp.float32),
    )(a0p, a1p, pp, dp, bt)


def _packed(a):
    return a.reshape(ACC_PACK, 128)


def kernel(x, edge_index, W1, b1, W2, b2, W3, b3):
    n = x.shape[0]
    e = edge_index.shape[1]
    gpt = _groups_per_tile(e)
    e_pad = gpt * NW * GE

    src = edge_index[0].astype(jnp.int32)
    dst = edge_index[1].astype(jnp.int32)
    pad = e_pad - e
    src_flat = jnp.concatenate([src, jnp.zeros((pad,), jnp.int32)])
    dst_flat = jnp.concatenate([dst, jnp.full((pad,), n, jnp.int32)])
    edges_grp = jnp.stack([src_flat.reshape(-1, GE),
                           dst_flat.reshape(-1, GE)], axis=1)

    eye = jnp.eye(PACK, dtype=jnp.float32)

    def kr(w):
        return jnp.kron(eye, jnp.pad(w, ((0, FEAT - w.shape[0]),
                                         (0, FEAT - w.shape[1]))))

    w1b = kr(W1)
    w2b = kr(W2)
    w3b = kr(W3)
    bcast = jnp.kron(eye, jnp.zeros((FEAT, FEAT), jnp.float32)
                     .at[0, :].set(1.0))
    b1t = jnp.tile(jnp.pad(b1, (0, FEAT - b1.shape[0])), PACK).reshape(1, 128)
    b2t = jnp.tile(jnp.pad(b2, (0, FEAT - b2.shape[0])), PACK).reshape(1, 128)
    b3t = jnp.tile(jnp.pad(b3, (0, FEAT - b3.shape[0])), PACK).reshape(1, 128)
    xp = jnp.pad(x, ((0, ACC_ROWS - n), (0, FEAT - x.shape[1]))
                 ).reshape(ACC_PACK, 128)

    zeros_f = jnp.zeros((ACC_ROWS, FEAT), jnp.float32)
    ones_blk = jnp.zeros((GE, FEAT), jnp.float32).at[:, 0].set(1.0)

    deg0, deg1 = _sc_degree(dst_flat, zeros_f, ones_blk, gpt)
    dp, p1p = _tc_stage_a(_packed(deg0), _packed(deg1), xp, w1b, bcast)

    a0, a1 = _sc_aggregate(edges_grp,
                           p1p.reshape(ACC_ROWS, FEAT), zeros_f, gpt)
    p2p = _tc_stage_mid(_packed(a0), _packed(a1), p1p, dp, b1t, w2b)

    a0, a1 = _sc_aggregate(edges_grp,
                           p2p.reshape(ACC_ROWS, FEAT), zeros_f, gpt)
    p3p = _tc_stage_mid(_packed(a0), _packed(a1), p2p, dp, b2t, w3b)

    a0, a1 = _sc_aggregate(edges_grp,
                           p3p.reshape(ACC_ROWS, FEAT), zeros_f, gpt)
    res = _tc_stage_last(_packed(a0), _packed(a1), p3p, dp, b3t)
    return res.reshape(ACC_ROWS, FEAT)[:n, : W3.shape[1]]


# speedup vs baseline: 1.0610x; 1.0610x over previous
import jax
import jax.numpy as jnp
from jax import lax
from jax.experimental import pallas as pl
from jax.experimental.pallas import tpu as pltpu
from jax.experimental.pallas import tpu_sc as plsc

NC = 2
NS = 16
NW = NC * NS
GE = 768
FEAT = 16
PACK = 8

STRIPE = 6256
ACC_ROWS = STRIPE * NS
ACC_PACK = ACC_ROWS // PACK


def _mesh():
    return plsc.VectorSubcoreMesh(core_axis_name="c", subcore_axis_name="s")


def _groups_per_tile(n_edges):
    g = -(-n_edges // (NW * GE))
    return -(-g // 6) * 6


def _acc_out_spec():
    return (jax.ShapeDtypeStruct((ACC_ROWS, FEAT), jnp.float32),
            jax.ShapeDtypeStruct((ACC_ROWS, FEAT), jnp.float32))


def _zero_and_barrier(z_hbm, acc_sh, s):
    pltpu.sync_copy(z_hbm.at[pl.ds(s * STRIPE, STRIPE)],
                    acc_sh.at[pl.ds(s * STRIPE, STRIPE)])


def _copy_out(acc_sh, out0, out1, c, s):
    @pl.when(c == 0)
    def _():
        pltpu.sync_copy(acc_sh.at[pl.ds(s * STRIPE, STRIPE)],
                        out0.at[pl.ds(s * STRIPE, STRIPE)])

    @pl.when(c == 1)
    def _():
        pltpu.sync_copy(acc_sh.at[pl.ds(s * STRIPE, STRIPE)],
                        out1.at[pl.ds(s * STRIPE, STRIPE)])


def _sc_degree(dst_flat, zeros_f, ones_blk, gpt):
    def body(dst_hbm, z_hbm, one_hbm, out0, out1, dstv0, dstv1, onev,
             acc_sh, ss0, ss1):
        c = lax.axis_index("c")
        s = lax.axis_index("s")
        _zero_and_barrier(z_hbm, acc_sh, s)
        pltpu.sync_copy(one_hbm, onev)
        plsc.subcore_barrier()
        base = (c * NS + s) * gpt
        dstv = (dstv0, dstv1)
        ss = (ss0, ss1)

        def steps(g, b):
            @pl.when(g >= 2)
            def _():
                pltpu.make_async_copy(onev, acc_sh.at[dstv[b]], ss[b]).wait()
            pltpu.sync_copy(dst_hbm.at[pl.ds((base + g) * GE, GE)], dstv[b])
            pltpu.async_copy(onev, acc_sh.at[dstv[b]], ss[b], add=True)

        def pair(i, carry):
            steps(2 * i, 0)
            steps(2 * i + 1, 1)
            return carry

        lax.fori_loop(0, gpt // 2, pair, 0)
        pltpu.make_async_copy(onev, acc_sh.at[dstv0], ss0).wait()
        pltpu.make_async_copy(onev, acc_sh.at[dstv1], ss1).wait()
        plsc.subcore_barrier()
        _copy_out(acc_sh, out0, out1, c, s)

    f = pl.kernel(
        body,
        out_type=_acc_out_spec(),
        mesh=_mesh(),
        scratch_types=[
            pltpu.VMEM((GE,), jnp.int32),
            pltpu.VMEM((GE,), jnp.int32),
            pltpu.VMEM((GE, FEAT), jnp.float32),
            pltpu.VMEM_SHARED((ACC_ROWS, FEAT), jnp.float32),
            pltpu.SemaphoreType.DMA,
            pltpu.SemaphoreType.DMA,
        ],
        compiler_params=pltpu.CompilerParams(use_tc_tiling_on_sc=False),
    )
    return f(dst_flat, zeros_f, ones_blk)


def _sc_aggregate(src_flat, dst_flat, p, zeros_f, gpt):
    def body(src_hbm, dst_hbm, p_hbm, z_hbm, out0, out1, srcv0, srcv1,
             dstv0, dstv1, rows0, rows1, acc_sh, gs0, gs1, ss0, ss1):
        c = lax.axis_index("c")
        s = lax.axis_index("s")
        _zero_and_barrier(z_hbm, acc_sh, s)
        plsc.subcore_barrier()
        base = (c * NS + s) * gpt
        srcv = (srcv0, srcv1)
        dstv = (dstv0, dstv1)
        rows = (rows0, rows1)
        gs = (gs0, gs1)
        ss = (ss0, ss1)

        def load(g, b):
            off = (base + g) * GE
            pltpu.sync_copy(src_hbm.at[pl.ds(off, GE)], srcv[b])
            pltpu.sync_copy(dst_hbm.at[pl.ds(off, GE)], dstv[b])

        def steps(g, b):
            nb = 1 - b
            @pl.when(g >= 1)
            def _():
                pltpu.make_async_copy(rows[nb], acc_sh.at[dstv[nb]],
                                      ss[nb]).wait()

            @pl.when(g + 1 < gpt)
            def _():
                load(g + 1, nb)
            pltpu.make_async_copy(p_hbm.at[srcv[b]], rows[b], gs[b]).wait()
            pltpu.async_copy(rows[b], acc_sh.at[dstv[b]], ss[b], add=True)

            @pl.when(g + 1 < gpt)
            def _():
                pltpu.async_copy(p_hbm.at[srcv[nb]], rows[nb], gs[nb])

        def pair(i, carry):
            steps(2 * i, 0)
            steps(2 * i + 1, 1)
            return carry

        load(0, 0)
        pltpu.async_copy(p_hbm.at[srcv0], rows0, gs0)
        lax.fori_loop(0, gpt // 2, pair, 0)
        pltpu.make_async_copy(rows1, acc_sh.at[dstv1], ss1).wait()
        plsc.subcore_barrier()
        _copy_out(acc_sh, out0, out1, c, s)

    f = pl.kernel(
        body,
        out_type=_acc_out_spec(),
        mesh=_mesh(),
        scratch_types=[
            pltpu.VMEM((GE,), jnp.int32),
            pltpu.VMEM((GE,), jnp.int32),
            pltpu.VMEM((GE,), jnp.int32),
            pltpu.VMEM((GE,), jnp.int32),
            pltpu.VMEM((GE, FEAT), jnp.float32),
            pltpu.VMEM((GE, FEAT), jnp.float32),
            pltpu.VMEM_SHARED((ACC_ROWS, FEAT), jnp.float32),
            pltpu.SemaphoreType.DMA,
            pltpu.SemaphoreType.DMA,
            pltpu.SemaphoreType.DMA,
            pltpu.SemaphoreType.DMA,
        ],
        compiler_params=pltpu.CompilerParams(use_tc_tiling_on_sc=False),
    )
    return f(src_flat, dst_flat, p, zeros_f)


_PBLK = 1024


def _pgrid():
    return (-(-ACC_PACK // _PBLK),)


def _pspec():
    return pl.BlockSpec((_PBLK, 128), lambda i: (i, 0))


def _fspec(r, c):
    return pl.BlockSpec((r, c), lambda i: (0, 0))


def _stage_a_body(a0, a1, xp, wb, bmat, d_out, p_out):
    deg = jnp.dot(a0[...] + a1[...], bmat[...],
                  preferred_element_type=jnp.float32) + 1.0
    dd = lax.rsqrt(deg)
    d_out[...] = dd
    xw = jnp.dot(xp[...], wb[...], preferred_element_type=jnp.float32)
    p_out[...] = dd * xw


def _stage_mid_body(a0, a1, p, d, bt, wb, out):
    dd = d[...]
    h = dd * (a0[...] + a1[...] + p[...]) + bt[...]
    out[...] = dd * jnp.dot(jnp.tanh(h), wb[...],
                            preferred_element_type=jnp.float32)


def _stage_last_body(a0, a1, p, d, bt, out):
    out[...] = d[...] * (a0[...] + a1[...] + p[...]) + bt[...]


def _tc_stage_a(a0p, a1p, xp, w1b, bmat):
    return pl.pallas_call(
        _stage_a_body,
        grid=_pgrid(),
        in_specs=[_pspec(), _pspec(), _pspec(),
                  _fspec(128, 128), _fspec(128, 128)],
        out_specs=[_pspec(), _pspec()],
        out_shape=[jax.ShapeDtypeStruct((ACC_PACK, 128), jnp.float32),
                   jax.ShapeDtypeStruct((ACC_PACK, 128), jnp.float32)],
    )(a0p, a1p, xp, w1b, bmat)


def _tc_stage_mid(a0p, a1p, pp, dp, bt, wbig):
    return pl.pallas_call(
        _stage_mid_body,
        grid=_pgrid(),
        in_specs=[_pspec(), _pspec(), _pspec(), _pspec(),
                  _fspec(1, 128), _fspec(128, 128)],
        out_specs=_pspec(),
        out_shape=jax.ShapeDtypeStruct((ACC_PACK, 128), jnp.float32),
    )(a0p, a1p, pp, dp, bt, wbig)


def _tc_stage_last(a0p, a1p, pp, dp, bt):
    return pl.pallas_call(
        _stage_last_body,
        grid=_pgrid(),
        in_specs=[_pspec(), _pspec(), _pspec(), _pspec(), _fspec(1, 128)],
        out_specs=_pspec(),
        out_shape=jax.ShapeDtypeStruct((ACC_PACK, 128), jnp.float32),
    )(a0p, a1p, pp, dp, bt)


def _packed(a):
    return a.reshape(ACC_PACK, 128)


def kernel(x, edge_index, W1, b1, W2, b2, W3, b3):
    n = x.shape[0]
    e = edge_index.shape[1]
    gpt = _groups_per_tile(e)
    e_pad = gpt * NW * GE

    src = edge_index[0].astype(jnp.int32)
    dst = edge_index[1].astype(jnp.int32)
    pad = e_pad - e
    src_flat = jnp.concatenate([src, jnp.zeros((pad,), jnp.int32)])
    dst_flat = jnp.concatenate([dst, jnp.full((pad,), n, jnp.int32)])

    eye = jnp.eye(PACK, dtype=jnp.float32)

    def kr(w):
        return jnp.kron(eye, jnp.pad(w, ((0, FEAT - w.shape[0]),
                                         (0, FEAT - w.shape[1]))))

    w1b = kr(W1)
    w2b = kr(W2)
    w3b = kr(W3)
    bcast = jnp.kron(eye, jnp.zeros((FEAT, FEAT), jnp.float32)
                     .at[0, :].set(1.0))
    b1t = jnp.tile(jnp.pad(b1, (0, FEAT - b1.shape[0])), PACK).reshape(1, 128)
    b2t = jnp.tile(jnp.pad(b2, (0, FEAT - b2.shape[0])), PACK).reshape(1, 128)
    b3t = jnp.tile(jnp.pad(b3, (0, FEAT - b3.shape[0])), PACK).reshape(1, 128)
    xp = jnp.pad(x, ((0, ACC_ROWS - n), (0, FEAT - x.shape[1]))
                 ).reshape(ACC_PACK, 128)

    zeros_f = jnp.zeros((ACC_ROWS, FEAT), jnp.float32)
    ones_blk = jnp.zeros((GE, FEAT), jnp.float32).at[:, 0].set(1.0)

    deg0, deg1 = _sc_degree(dst_flat, zeros_f, ones_blk, gpt)
    dp, p1p = _tc_stage_a(_packed(deg0), _packed(deg1), xp, w1b, bcast)

    a0, a1 = _sc_aggregate(src_flat, dst_flat,
                           p1p.reshape(ACC_ROWS, FEAT), zeros_f, gpt)
    p2p = _tc_stage_mid(_packed(a0), _packed(a1), p1p, dp, b1t, w2b)

    a0, a1 = _sc_aggregate(src_flat, dst_flat,
                           p2p.reshape(ACC_ROWS, FEAT), zeros_f, gpt)
    p3p = _tc_stage_mid(_packed(a0), _packed(a1), p2p, dp, b2t, w3b)

    a0, a1 = _sc_aggregate(src_flat, dst_flat,
                           p3p.reshape(ACC_ROWS, FEAT), zeros_f, gpt)
    res = _tc_stage_last(_packed(a0), _packed(a1), p3p, dp, b3t)
    return res.reshape(ACC_ROWS, FEAT)[:n, : W3.shape[1]]
